# Initial kernel scaffold; baseline (speedup 1.0000x reference)
import jax
import jax.numpy as jnp
from jax.experimental import pallas as pl


def kernel(user_indices, pos_item_indices, neg_item_indices, user_table, item_table, adj_rows, adj_cols, adj_vals):
    raise NotImplementedError("write your pallas kernel here")


# speedup vs baseline: 2.3195x; 2.3195x over previous
import functools

import jax
import jax.numpy as jnp
from jax import lax
from jax.experimental import pallas as pl
from jax.experimental.pallas import tpu as pltpu
from jax.experimental.pallas import tpu_sc as plsc

_NUM_USERS = 25000
_NUM_ITEMS = 25000
_N = _NUM_USERS + _NUM_ITEMS
_NP = 50048
_HALF = 32
_E = 800000
_LAYERS = 3
_WD = 1e-4
_B = 16384

_NC = 2
_NS = 16
_LANES = 16

_CK = 80
_BCK = 128
_WB = 184


def _make_sc_prop(n_pad, half, n_edges, n_batch, ck, bck, wb, n_layers,
                  interpret=False):
    nc, ns, lanes = _NC, _NS, _LANES
    ept = n_edges // ns
    nec = ept // ck
    npt = n_pad // ns
    nwb = npt // wb
    bpt = n_batch // ns
    nbc = bpt // bck
    assert ept * ns == n_edges and nec * ck == ept
    assert npt * ns == n_pad and nwb * wb == npt and npt % 8 == 0 and wb % 8 == 0
    assert bpt * ns == n_batch and nbc * bck == bpt

    mesh = plsc.VectorSubcoreMesh(core_axis_name="c", subcore_axis_name="s",
                                  num_cores=nc, num_subcores=ns)

    @functools.partial(
        pl.kernel,
        out_type=(
            jax.ShapeDtypeStruct((nc * n_pad, half), jnp.float32),
            jax.ShapeDtypeStruct((nc * n_pad, half), jnp.float32),
            jax.ShapeDtypeStruct((3, nc * n_batch, half), jnp.float32),
            jax.ShapeDtypeStruct((3, nc * n_batch, half), jnp.float32),
        ),
        mesh=mesh,
        scratch_types=[
            pltpu.VMEM_SHARED((n_pad, half), jnp.float32),
            pltpu.VMEM((ck, half), jnp.float32),
            pltpu.VMEM((ck,), jnp.int32),
            pltpu.VMEM((ck,), jnp.int32),
            pltpu.VMEM((ck,), jnp.float32),
            pltpu.VMEM((ck,), jnp.int32),
            pltpu.VMEM((wb, half), jnp.float32),
            pltpu.VMEM((wb, half), jnp.float32),
            pltpu.VMEM((wb, half), jnp.float32),
            pltpu.VMEM((bck,), jnp.int32),
            pltpu.VMEM((bck,), jnp.int32),
            pltpu.VMEM((bck, half), jnp.float32),
        ],
        compiler_params=pltpu.CompilerParams(use_tc_tiling_on_sc=False),
        interpret=interpret,
    )
    def sc_prop(emb0, cols1d, rows1d, vals1d, ui, pi, ni,
                cur, summ, gprop, gorig,
                acc_s, gbuf, ccol, crow, cval, cadj, t1, t2, zbuf,
                bidx, badj, gbuf2):
        c = lax.axis_index("c")
        s = lax.axis_index("s")
        c_n = c * n_pad
        zeros16 = jnp.zeros((lanes,), jnp.float32)

        @pl.loop(0, wb)
        def _(rr):
            zbuf[rr, pl.ds(0, lanes)] = zeros16
            zbuf[rr, pl.ds(lanes, lanes)] = zeros16

        @pl.loop(0, nwb)
        def _(k):
            pltpu.sync_copy(zbuf, acc_s.at[pl.ds(s * npt + k * wb, wb)])

        plsc.subcore_barrier()

        for layer in range(n_layers):
            src = emb0 if layer == 0 else cur

            @pl.loop(0, nec)
            def _(j):
                e0 = s * ept + j * ck
                pltpu.sync_copy(cols1d.at[pl.ds(e0, ck)], ccol)
                pltpu.sync_copy(rows1d.at[pl.ds(e0, ck)], crow)
                pltpu.sync_copy(vals1d.at[pl.ds(e0, ck)], cval)
                for k in range(ck // lanes):
                    cadj[pl.ds(k * lanes, lanes)] = (
                        ccol[pl.ds(k * lanes, lanes)] + c_n)
                pltpu.sync_copy(src.at[cadj], gbuf)

                @pl.loop(0, ck // lanes)
                def _(g):
                    vv = cval[pl.ds(g * lanes, lanes)]
                    for i in range(lanes):
                        e = g * lanes + i
                        v = vv[i]
                        gbuf[e, pl.ds(0, lanes)] = gbuf[e, pl.ds(0, lanes)] * v
                        gbuf[e, pl.ds(lanes, lanes)] = (
                            gbuf[e, pl.ds(lanes, lanes)] * v)

                pltpu.sync_copy(gbuf, acc_s.at[crow], add=True)

            plsc.subcore_barrier()

            sum_src = emb0 if layer == 0 else summ

            @pl.loop(0, nwb)
            def _(k):
                r0 = s * npt + k * wb
                g0 = c_n + r0
                pltpu.sync_copy(acc_s.at[pl.ds(r0, wb)], t1)
                pltpu.sync_copy(zbuf, acc_s.at[pl.ds(r0, wb)])
                pltpu.sync_copy(t1, cur.at[pl.ds(g0, wb)])
                pltpu.sync_copy(sum_src.at[pl.ds(g0, wb)], t2)

                @pl.loop(0, wb)
                def _(rr):
                    t2[rr, pl.ds(0, lanes)] = (
                        t2[rr, pl.ds(0, lanes)] + t1[rr, pl.ds(0, lanes)])
                    t2[rr, pl.ds(lanes, lanes)] = (
                        t2[rr, pl.ds(lanes, lanes)] + t1[rr, pl.ds(lanes, lanes)])

                pltpu.sync_copy(t2, summ.at[pl.ds(g0, wb)])

            plsc.subcore_barrier()

        @pl.loop(0, nbc)
        def _(j):
            base = s * bpt + j * bck
            for a, idx1d in enumerate((ui, pi, ni)):
                pltpu.sync_copy(idx1d.at[pl.ds(base, bck)], bidx)
                for k in range(bck // lanes):
                    badj[pl.ds(k * lanes, lanes)] = (
                        bidx[pl.ds(k * lanes, lanes)] + c_n)
                pltpu.sync_copy(summ.at[badj], gbuf2)
                pltpu.sync_copy(
                    gbuf2, gprop.at[a, pl.ds(c * n_batch + base, bck)])
                pltpu.sync_copy(emb0.at[badj], gbuf2)
                pltpu.sync_copy(
                    gbuf2, gorig.at[a, pl.ds(c * n_batch + base, bck)])

    return sc_prop


_sc_prop = _make_sc_prop(_NP, _HALF, _E, _B, _CK, _BCK, _WB, _LAYERS)


_GRP = 128 // _HALF
_NR = _NC * _B // _GRP
_NR2 = _NR // _NC


def _tc_finish(gprop_ref, gorig_ref, out_ref):
    u = gprop_ref[0]
    p = gprop_ref[1]
    q = gprop_ref[2]
    prod = u * (p - q)
    r_iota = lax.broadcasted_iota(jnp.int32, (128, 128), 0)
    c_iota = lax.broadcasted_iota(jnp.int32, (128, 128), 1)
    m = jnp.where((r_iota // _HALF) == c_iota, 1.0, 0.0)
    s = jnp.dot(prod, m, preferred_element_type=jnp.float32)
    d4 = (s[:_NR2, :] + s[_NR2:, :]) * (1.0 / ((_LAYERS + 1) ** 2))
    z = -d4
    sp = jnp.maximum(z, 0.0) + jnp.log1p(jnp.exp(-jnp.abs(z)))
    col = lax.broadcasted_iota(jnp.int32, (_NR2, 128), 1)
    bpr = jnp.sum(jnp.where(col < _GRP, sp, 0.0)) * (1.0 / _B)
    og = gorig_ref[...]
    reg = jnp.sum(og * og) * (1.0 / _B)
    loss = bpr + _WD * reg
    out_ref[...] = jnp.concatenate(
        [jnp.full((1, 128), loss, jnp.float32),
         jnp.full((1, 128), bpr, jnp.float32),
         jnp.full((1, 128), reg, jnp.float32),
         jnp.zeros((5, 128), jnp.float32)], axis=0)


def kernel(user_indices, pos_item_indices, neg_item_indices,
           user_table, item_table, adj_rows, adj_cols, adj_vals):
    all0 = jnp.concatenate([user_table, item_table], axis=0)
    pad = ((0, _NP - _N), (0, 0))
    emb0 = jnp.concatenate(
        [jnp.pad(all0[:, :_HALF], pad), jnp.pad(all0[:, _HALF:], pad)], axis=0)
    ui = user_indices.astype(jnp.int32)
    pi = pos_item_indices.astype(jnp.int32) + _NUM_USERS
    ni = neg_item_indices.astype(jnp.int32) + _NUM_USERS
    cur, summ, gprop, gorig = _sc_prop(
        emb0, adj_cols, adj_rows, adj_vals, ui, pi, ni)
    out = pl.pallas_call(
        _tc_finish,
        out_shape=jax.ShapeDtypeStruct((8, 128), jnp.float32),
    )(gprop.reshape(3, _NR, 128), gorig.reshape(3, _NR, 128))
    return (out[0, 0], out[1, 0], out[2, 0])
